# baseline (speedup 1.0000x reference)
import functools

import jax
import jax.numpy as jnp
from jax import lax
from jax.experimental import pallas as pl
from jax.experimental.pallas import tpu as pltpu
from jax.experimental.pallas import tpu_sc as plsc

LATENT = 16
LANES = 16
PACK = 128 // LATENT
CHUNK = 256


def _gmf_sc(user, item, user_table, item_table):
    B = user.shape[0]
    info = plsc.get_sparse_core_info()
    NC, NS = info.num_cores, info.num_subcores
    NW = NC * NS
    assert B % (CHUNK * NW) == 0
    b_per_w = B // NW
    n_chunks = b_per_w // CHUNK

    ut2 = user_table.reshape(user_table.shape[0] // PACK, 128)
    it2 = item_table.reshape(item_table.shape[0] // PACK, 128)

    mesh = plsc.VectorSubcoreMesh(core_axis_name="c", subcore_axis_name="s")

    @functools.partial(
        pl.kernel,
        mesh=mesh,
        compiler_params=pltpu.CompilerParams(needs_layout_passes=False),
        out_type=jax.ShapeDtypeStruct((B,), jnp.float32),
        scratch_types=[
            pltpu.VMEM((b_per_w,), jnp.int32),
            pltpu.VMEM((b_per_w,), jnp.int32),
            pltpu.VMEM((CHUNK,), jnp.int32),
            pltpu.VMEM((CHUNK,), jnp.int32),
            pltpu.VMEM((CHUNK, 128), jnp.float32),
            pltpu.VMEM((CHUNK, 128), jnp.float32),
            pltpu.VMEM((b_per_w,), jnp.float32),
            pltpu.SemaphoreType.DMA,
        ],
    )
    def gmf_kernel(user_hbm, item_hbm, ut_hbm, it_hbm, out_hbm,
                   uidx_v, iidx_v, urow_v, irow_v, ubuf_v, ibuf_v, out_v,
                   sem):
        wid = lax.axis_index("s") * NC + lax.axis_index("c")
        base = wid * b_per_w
        pltpu.sync_copy(user_hbm.at[pl.ds(base, b_per_w)], uidx_v)
        pltpu.sync_copy(item_hbm.at[pl.ds(base, b_per_w)], iidx_v)

        lane = lax.iota(jnp.int32, LANES)

        def chunk_body(c, carry):
            off = c * CHUNK

            def idx_body(g, carry2):
                s = g * LANES
                urow_v[pl.ds(s, LANES)] = (
                    uidx_v[pl.ds(off + s, LANES)] >> 3)
                irow_v[pl.ds(s, LANES)] = (
                    iidx_v[pl.ds(off + s, LANES)] >> 3)
                return carry2

            lax.fori_loop(0, CHUNK // LANES, idx_body, 0)

            cp_u = pltpu.async_copy(ut_hbm.at[urow_v], ubuf_v, sem)
            cp_i = pltpu.async_copy(it_hbm.at[irow_v], ibuf_v, sem)
            cp_u.wait()
            cp_i.wait()

            def group_body(g, carry2):
                s = g * LANES
                usub = (uidx_v[pl.ds(off + s, LANES)] & 7) * LATENT
                isub = (iidx_v[pl.ds(off + s, LANES)] & 7) * LATENT
                acc = jnp.zeros((LANES,), jnp.float32)
                for d in range(LATENT):
                    uc = plsc.load_gather(ubuf_v, [lane + s, usub + d])
                    ic = plsc.load_gather(ibuf_v, [lane + s, isub + d])
                    acc = acc + uc * ic
                z = jnp.exp(-jnp.abs(acc))
                r = jnp.where(acc >= 0.0, 1.0 / (1.0 + z), z / (1.0 + z))
                out_v[pl.ds(off + s, LANES)] = r
                return carry2

            lax.fori_loop(0, CHUNK // LANES, group_body, 0)
            return carry

        lax.fori_loop(0, n_chunks, chunk_body, 0)
        pltpu.sync_copy(out_v, out_hbm.at[pl.ds(base, b_per_w)])

    return gmf_kernel(user, item, ut2, it2)


def kernel(user, item, user_table, item_table):
    return _gmf_sc(user, item, user_table, item_table)


# speedup vs baseline: 6.1556x; 6.1556x over previous
import functools

import jax
import jax.numpy as jnp
from jax import lax
from jax.experimental import pallas as pl
from jax.experimental.pallas import tpu as pltpu
from jax.experimental.pallas import tpu_sc as plsc

LATENT = 16
LANES = 16


def _gmf_sc(user, item, user_table, item_table):
    B = user.shape[0]
    info = plsc.get_sparse_core_info()
    NC, NS = info.num_cores, info.num_subcores
    NW = NC * NS
    assert B % (LANES * NW) == 0
    b_per_w = B // NW
    n_groups = b_per_w // LANES

    mesh = plsc.VectorSubcoreMesh(core_axis_name="c", subcore_axis_name="s")

    @functools.partial(
        pl.kernel,
        mesh=mesh,
        compiler_params=pltpu.CompilerParams(needs_layout_passes=False),
        out_type=jax.ShapeDtypeStruct((B,), jnp.float32),
        scratch_types=[
            pltpu.VMEM((b_per_w,), jnp.int32),
            pltpu.VMEM((b_per_w,), jnp.int32),
            pltpu.VMEM((LANES, LATENT, 128), jnp.float32),
            pltpu.VMEM((LANES, LATENT, 128), jnp.float32),
            pltpu.VMEM((b_per_w,), jnp.float32),
            pltpu.SemaphoreType.DMA,
        ],
    )
    def gmf_kernel(user_hbm, item_hbm, ut_hbm, it_hbm, out_hbm,
                   uidx_v, iidx_v, uslots_v, islots_v, out_v, sem):
        wid = lax.axis_index("s") * NC + lax.axis_index("c")
        base = wid * b_per_w
        pltpu.sync_copy(user_hbm.at[pl.ds(base, b_per_w)], uidx_v)
        pltpu.sync_copy(item_hbm.at[pl.ds(base, b_per_w)], iidx_v)

        lane = lax.iota(jnp.int32, LANES)

        def group_body(g, carry):
            s = g * LANES
            ru = uidx_v[pl.ds(s, LANES)]
            ri = iidx_v[pl.ds(s, LANES)]
            bu_vec = (ru >> 7) * 128
            bi_vec = (ri >> 7) * 128

            for j in range(LANES):
                bu = pl.multiple_of(bu_vec[j], 128)
                bi = pl.multiple_of(bi_vec[j], 128)
                pltpu.async_copy(
                    ut_hbm.at[:, pl.ds(bu, 128)], uslots_v.at[j], sem)
                pltpu.async_copy(
                    it_hbm.at[:, pl.ds(bi, 128)], islots_v.at[j], sem)

            def drain(j, carry2):
                pltpu.make_async_copy(
                    ut_hbm.at[:, pl.ds(0, 128)], uslots_v.at[0], sem).wait()
                pltpu.make_async_copy(
                    it_hbm.at[:, pl.ds(0, 128)], islots_v.at[0], sem).wait()
                return carry2

            lax.fori_loop(0, LANES, drain, 0)

            ru = uidx_v[pl.ds(s, LANES)]
            ri = iidx_v[pl.ds(s, LANES)]
            lu = ru & 127
            li = ri & 127
            acc = jnp.zeros((LANES,), jnp.float32)
            for d in range(LATENT):
                dd = jnp.full((LANES,), d, jnp.int32)
                uc = plsc.load_gather(uslots_v, [lane, dd, lu])
                ic = plsc.load_gather(islots_v, [lane, dd, li])
                acc = acc + uc * ic
            z = jnp.exp(-jnp.abs(acc))
            r = jnp.where(acc >= 0.0, 1.0 / (1.0 + z), z / (1.0 + z))
            out_v[pl.ds(s, LANES)] = r
            return carry

        lax.fori_loop(0, n_groups, group_body, 0)
        pltpu.sync_copy(out_v, out_hbm.at[pl.ds(base, b_per_w)])

    return gmf_kernel(user, item, user_table.T, item_table.T)


def kernel(user, item, user_table, item_table):
    return _gmf_sc(user, item, user_table, item_table)


# speedup vs baseline: 6.2456x; 1.0146x over previous
import functools

import jax
import jax.numpy as jnp
from jax import lax
from jax.experimental import pallas as pl
from jax.experimental.pallas import tpu as pltpu
from jax.experimental.pallas import tpu_sc as plsc

LATENT = 16
LANES = 16
HALF = 8


def _gmf_sc(user, item, user_table, item_table):
    B = user.shape[0]
    info = plsc.get_sparse_core_info()
    NC, NS = info.num_cores, info.num_subcores
    NW = NC * NS
    assert B % (LANES * NW) == 0
    b_per_w = B // NW
    n_groups = b_per_w // LANES

    mesh = plsc.VectorSubcoreMesh(core_axis_name="c", subcore_axis_name="s")

    @functools.partial(
        pl.kernel,
        mesh=mesh,
        compiler_params=pltpu.CompilerParams(needs_layout_passes=False),
        out_type=jax.ShapeDtypeStruct((B,), jnp.float32),
        scratch_types=[
            pltpu.VMEM((b_per_w,), jnp.int32),
            pltpu.VMEM((b_per_w,), jnp.int32),
            pltpu.VMEM((LANES, HALF, 128), jnp.float32),
            pltpu.VMEM((LANES, HALF, 128), jnp.float32),
            pltpu.VMEM((LANES, HALF, 128), jnp.float32),
            pltpu.VMEM((LANES, HALF, 128), jnp.float32),
            pltpu.VMEM((b_per_w,), jnp.float32),
            pltpu.SemaphoreType.DMA,
            pltpu.SemaphoreType.DMA,
        ],
    )
    def gmf_kernel(user_hbm, item_hbm, ut_hbm, it_hbm, out_hbm,
                   uidx_v, iidx_v, uslots0, islots0, uslots1, islots1,
                   out_v, sem0, sem1):
        wid = lax.axis_index("s") * NC + lax.axis_index("c")
        base = wid * b_per_w
        pltpu.sync_copy(user_hbm.at[pl.ds(base, b_per_w)], uidx_v)
        pltpu.sync_copy(item_hbm.at[pl.ds(base, b_per_w)], iidx_v)

        lane = lax.iota(jnp.int32, LANES)

        def fire(g, h, uslots, islots, sem):
            s = g * LANES
            ru = uidx_v[pl.ds(s, LANES)]
            ri = iidx_v[pl.ds(s, LANES)]
            bu_vec = (ru >> 7) * 128
            bi_vec = (ri >> 7) * 128
            row = pl.ds(h * HALF, HALF)
            for j in range(LANES):
                bu = pl.multiple_of(bu_vec[j], 128)
                bi = pl.multiple_of(bi_vec[j], 128)
                pltpu.async_copy(
                    ut_hbm.at[row, pl.ds(bu, 128)], uslots.at[j], sem)
                pltpu.async_copy(
                    it_hbm.at[row, pl.ds(bi, 128)], islots.at[j], sem)

        def drain(j, carry, *, uslots, islots, sem):
            pltpu.make_async_copy(
                ut_hbm.at[pl.ds(0, HALF), pl.ds(0, 128)],
                uslots.at[0], sem).wait()
            pltpu.make_async_copy(
                it_hbm.at[pl.ds(0, HALF), pl.ds(0, 128)],
                islots.at[0], sem).wait()
            return carry

        def reduce_half(g, uslots, islots):
            s = g * LANES
            ru = uidx_v[pl.ds(s, LANES)]
            ri = iidx_v[pl.ds(s, LANES)]
            lu = ru & 127
            li = ri & 127
            acc = jnp.zeros((LANES,), jnp.float32)
            for d in range(HALF):
                dd = jnp.full((LANES,), d, jnp.int32)
                uc = plsc.load_gather(uslots, [lane, dd, lu])
                ic = plsc.load_gather(islots, [lane, dd, li])
                acc = acc + uc * ic
            return acc

        fire(0, 0, uslots0, islots0, sem0)

        def group_body(g, carry):
            fire(g, 1, uslots1, islots1, sem1)
            lax.fori_loop(0, LANES, functools.partial(
                drain, uslots=uslots0, islots=islots0, sem=sem0), 0)
            acc0 = reduce_half(g, uslots0, islots0)

            @pl.when(g + 1 < n_groups)
            def _():
                fire(g + 1, 0, uslots0, islots0, sem0)

            lax.fori_loop(0, LANES, functools.partial(
                drain, uslots=uslots1, islots=islots1, sem=sem1), 0)
            acc1 = reduce_half(g, uslots1, islots1)

            acc = acc0 + acc1
            z = jnp.exp(-jnp.abs(acc))
            r = jnp.where(acc >= 0.0, 1.0 / (1.0 + z), z / (1.0 + z))
            out_v[pl.ds(g * LANES, LANES)] = r
            return carry

        lax.fori_loop(0, n_groups, group_body, 0)
        pltpu.sync_copy(out_v, out_hbm.at[pl.ds(base, b_per_w)])

    return gmf_kernel(user, item, user_table.T, item_table.T)


def kernel(user, item, user_table, item_table):
    return _gmf_sc(user, item, user_table, item_table)
